# baseline (speedup 1.0000x reference)
import functools

import jax
import jax.numpy as jnp
from jax import lax
from jax.experimental import pallas as pl
from jax.experimental.pallas import tpu as pltpu
from jax.experimental.pallas import tpu_sc as plsc

_SUB = 128
_NB = 4


@functools.partial(jax.jit, static_argnames=("hist", "hist_pad"))
def _sc_embed(idx_grouped, table_padded, hist, hist_pad):
    nw, sub, idx_w = idx_grouped.shape
    two_d = table_padded.shape[1]
    info = plsc.get_sparse_core_info()
    nc = info.num_cores

    mesh = plsc.VectorSubcoreMesh(core_axis_name="c", subcore_axis_name="s")

    @functools.partial(
        pl.kernel,
        out_type=jax.ShapeDtypeStruct((nw * sub * hist_pad, two_d), jnp.float32),
        mesh=mesh,
        scratch_types=[
            pltpu.VMEM((sub, idx_w), jnp.int32),
            [pltpu.VMEM((hist_pad, two_d), jnp.float32)] * _NB,
            [pltpu.SemaphoreType.DMA] * _NB,
            [pltpu.SemaphoreType.DMA] * _NB,
        ],
        compiler_params=pltpu.CompilerParams(
            use_tc_tiling_on_sc=True, needs_layout_passes=False
        ),
    )
    def body(idx_hbm, tbl_hbm, out_hbm, idx_v, stages, gsems, wsems):
        wid = lax.axis_index("s") * nc + lax.axis_index("c")
        pltpu.sync_copy(idx_hbm.at[wid], idx_v)

        def fire(i, b):
            pltpu.async_copy(
                tbl_hbm.at[idx_v.at[i, pl.ds(0, hist_pad)]], stages[b], gsems[b]
            )

        def drain(i, b):
            pltpu.make_async_copy(
                tbl_hbm.at[idx_v.at[i, pl.ds(0, hist_pad)]], stages[b], gsems[b]
            ).wait()
            pltpu.async_copy(
                stages[b],
                out_hbm.at[pl.ds((wid * sub + i) * hist_pad, hist_pad)],
                wsems[b],
            )

        def wait_write(i, b):
            pltpu.make_async_copy(
                stages[b],
                out_hbm.at[pl.ds((wid * sub + i) * hist_pad, hist_pad)],
                wsems[b],
            ).wait()

        for b in range(_NB):
            fire(b, b)

        def step(t, carry):
            for b in range(_NB):
                i = t * _NB + b
                drain(i, b)
                nxt = i + _NB

                @pl.when(nxt < sub)
                def _refill():
                    wait_write(i, b)
                    fire(nxt, b)

            return carry

        lax.fori_loop(0, sub // _NB, step, 0)
        for b in range(_NB):
            wait_write(sub - _NB + b, b)

    return body(idx_grouped, table_padded)


def kernel(input_ids, embedding_weight):
    batch, hist = input_ids.shape
    vocab, embed_dim = embedding_weight.shape
    hist_pad = -(-hist // 8) * 8

    info = plsc.get_sparse_core_info()
    nw = info.num_cores * info.num_subcores

    table_padded = jnp.pad(embedding_weight, ((0, 0), (0, 128 - embed_dim)))
    idx_grouped = jnp.pad(
        input_ids.astype(jnp.int32).reshape(nw, batch // nw, hist),
        ((0, 0), (0, 0), (0, 128 - hist)),
    )
    out = _sc_embed(idx_grouped, table_padded, hist, hist_pad)
    out3 = out.reshape(batch, hist_pad, 2 * embed_dim)
    return out3[:, :hist, :embed_dim]
